# baseline (speedup 1.0000x reference)
import functools

import jax
import jax.numpy as jnp
from jax import lax
from jax.experimental import pallas as pl
from jax.experimental.pallas import tpu as pltpu
from jax.experimental.pallas import tpu_sc as plsc

N = 10000
E = 320000
DIN = 128
H = 64
B = 512

NPAD = 10240
NC, NS = 2, 16
NW = NC * NS
CH = 128
NBUF = 4
FAST_CORE = 1
CPT_F = 124
CPT_S = 36
CPTMAX = CPT_F
EPAD = NS * (CPT_F + CPT_S) * CH
ROWS_PER_TILE = NPAD // NS
NODES_PER_TILE = NPAD // NW
DW = 16
BPAD = 640
BROWS = BPAD // NS

_mesh = plsc.VectorSubcoreMesh(core_axis_name="c", subcore_axis_name="s")


def _zero_vmem_2d(ref, nrows, width):
    z = jnp.zeros((16,), jnp.float32)

    def body(j, _):
        for q in range(width // 16):
            ref[j, pl.ds(q * 16, 16)] = z
        return 0

    lax.fori_loop(0, nrows, body, 0)


@functools.partial(
    pl.kernel,
    out_type=jax.ShapeDtypeStruct((NC, NPAD, DW), jnp.float32),
    mesh=_mesh,
    compiler_params=pltpu.CompilerParams(use_tc_tiling_on_sc=False, needs_layout_passes=False),
    scratch_types=[
        pltpu.VMEM((CPTMAX, CH), jnp.int32),
        pltpu.VMEM((CH, DW), jnp.float32),
        pltpu.VMEM((CH, DW), jnp.float32),
        pltpu.VMEM_SHARED((NPAD, DW), jnp.float32),
    ],
)
def _sc_deg(dst_hbm, out_hbm, dstv, onesv, zbuf, acc):
    c = lax.axis_index("c")
    s = lax.axis_index("s")
    wid = c * NS + s

    onehot = jnp.where(lax.iota(jnp.int32, 16) == 0, 1.0, 0.0).astype(jnp.float32)
    z = jnp.zeros((16,), jnp.float32)

    def init_body(j, _):
        onesv[j, pl.ds(0, 16)] = onehot
        zbuf[j, pl.ds(0, 16)] = z
        return 0

    lax.fori_loop(0, CH, init_body, 0)

    base = s * ROWS_PER_TILE
    for r in range(ROWS_PER_TILE // CH):
        pltpu.sync_copy(zbuf, acc.at[pl.ds(base + r * CH, CH)])
    plsc.subcore_barrier()

    pltpu.sync_copy(dst_hbm.at[wid], dstv)

    def edge_body(j, _):
        pltpu.sync_copy(onesv, acc.at[dstv.at[j]], add=True)
        return 0

    @pl.when(c == FAST_CORE)
    def _():
        lax.fori_loop(0, CPT_F, edge_body, 0)

    @pl.when(c != FAST_CORE)
    def _():
        lax.fori_loop(0, CPT_S, edge_body, 0)

    plsc.subcore_barrier()

    pltpu.sync_copy(acc.at[pl.ds(base, ROWS_PER_TILE)],
                    out_hbm.at[c, pl.ds(base, ROWS_PER_TILE)])


@functools.partial(
    pl.kernel,
    out_type=jax.ShapeDtypeStruct((NC, NPAD, H), jnp.float32),
    mesh=_mesh,
    compiler_params=pltpu.CompilerParams(use_tc_tiling_on_sc=False, needs_layout_passes=False),
    scratch_types=[
        pltpu.VMEM((CPTMAX, CH), jnp.int32),
        pltpu.VMEM((CPTMAX, CH), jnp.int32),
        pltpu.VMEM((NBUF, CH, H), jnp.float32),
        pltpu.VMEM((CH, H), jnp.float32),
        pltpu.VMEM_SHARED((NPAD, H), jnp.float32),
    ] + [pltpu.SemaphoreType.DMA] * NBUF,
)
def _sc_agg(hp_hbm, src_hbm, dst_hbm, out_hbm, srcv, dstv, rows, zbuf, acc,
            *sems):
    c = lax.axis_index("c")
    s = lax.axis_index("s")
    wid = c * NS + s

    _zero_vmem_2d(zbuf, CH, H)
    base = s * ROWS_PER_TILE
    for r in range(ROWS_PER_TILE // CH):
        pltpu.sync_copy(zbuf, acc.at[pl.ds(base + r * CH, CH)])
    plsc.subcore_barrier()

    pltpu.sync_copy(src_hbm.at[wid], srcv)
    pltpu.sync_copy(dst_hbm.at[wid], dstv)

    @pl.when(c == FAST_CORE)
    def _():
        for b in range(NBUF):
            pltpu.async_copy(hp_hbm.at[srcv.at[b]], rows.at[b], sems[b])

        def group_body(g, _):
            for b in range(NBUF):
                j = g * NBUF + b
                pltpu.make_async_copy(hp_hbm.at[srcv.at[j]], rows.at[b],
                                      sems[b]).wait()
                pltpu.sync_copy(rows.at[b], acc.at[dstv.at[j]], add=True)
                pltpu.async_copy(hp_hbm.at[srcv.at[j + NBUF]], rows.at[b],
                                 sems[b])
            return 0

        lax.fori_loop(0, CPT_F // NBUF - 1, group_body, 0)
        for b in range(NBUF):
            j = CPT_F - NBUF + b
            pltpu.make_async_copy(hp_hbm.at[srcv.at[j]], rows.at[b],
                                  sems[b]).wait()
            pltpu.sync_copy(rows.at[b], acc.at[dstv.at[j]], add=True)

    @pl.when(c != FAST_CORE)
    def _():
        def edge_body(j, _):
            pltpu.async_copy(hp_hbm.at[srcv.at[j]], rows.at[0], sems[0]).wait()
            pltpu.sync_copy(rows.at[0], acc.at[dstv.at[j]], add=True)
            return 0

        lax.fori_loop(0, CPT_S, edge_body, 0)

    plsc.subcore_barrier()

    pltpu.sync_copy(acc.at[pl.ds(base, ROWS_PER_TILE)],
                    out_hbm.at[c, pl.ds(base, ROWS_PER_TILE)])


@functools.partial(
    pl.kernel,
    out_type=(
        jax.ShapeDtypeStruct((NC, BPAD, H), jnp.float32),
        jax.ShapeDtypeStruct((NW, BPAD, H), jnp.float32),
        jax.ShapeDtypeStruct((NW, 1, BPAD), jnp.float32),
    ),
    mesh=_mesh,
    compiler_params=pltpu.CompilerParams(use_tc_tiling_on_sc=False, needs_layout_passes=False),
    scratch_types=[
        pltpu.VMEM((NODES_PER_TILE, H), jnp.float32),
        pltpu.VMEM((NODES_PER_TILE, H), jnp.float32),
        pltpu.VMEM((NODES_PER_TILE, H), jnp.float32),
        pltpu.VMEM((NODES_PER_TILE,), jnp.float32),
        pltpu.VMEM((NODES_PER_TILE // 64, 64), jnp.int32),
        pltpu.VMEM((H,), jnp.float32),
        pltpu.VMEM((64, H), jnp.float32),
        pltpu.VMEM((BPAD, H), jnp.float32),
        pltpu.VMEM((1, BPAD), jnp.float32),
        pltpu.VMEM_SHARED((BPAD, H), jnp.float32),
    ],
)
def _sc_pool(p_hbm, hp_hbm, dinv_hbm, batch_hbm, b4_hbm,
             sum_hbm, max_hbm, cnt_hbm,
             p0v, p1v, hpv, dinvv, batv, b4v, rowsb, maxp, cntp, sacc):
    c = lax.axis_index("c")
    s = lax.axis_index("s")
    wid = c * NS + s
    nbase = wid * NODES_PER_TILE

    _zero_vmem_2d(maxp, BPAD, H)
    z16 = jnp.zeros((16,), jnp.float32)
    for q in range(BPAD // 16):
        cntp[0, pl.ds(q * 16, 16)] = z16
    _zero_vmem_2d(rowsb, 64, H)
    pltpu.sync_copy(rowsb.at[pl.ds(0, BROWS)], sacc.at[pl.ds(s * BROWS, BROWS)])
    plsc.subcore_barrier()

    pltpu.sync_copy(p_hbm.at[0, pl.ds(nbase, NODES_PER_TILE)], p0v)
    pltpu.sync_copy(p_hbm.at[1, pl.ds(nbase, NODES_PER_TILE)], p1v)
    pltpu.sync_copy(hp_hbm.at[pl.ds(nbase, NODES_PER_TILE)], hpv)
    pltpu.sync_copy(dinv_hbm.at[pl.ds(nbase, NODES_PER_TILE)], dinvv)
    pltpu.sync_copy(batch_hbm.at[wid], batv)
    pltpu.sync_copy(b4_hbm, b4v)

    colbase = lax.iota(jnp.int32, 16)
    one16 = jnp.full((16,), 1.0, jnp.float32)
    zi16 = jnp.zeros((16,), jnp.int32)

    def chunk_body(ci, _):
        def node_body(i, _):
            n = ci * 64 + i
            n16 = jnp.full((16,), n, jnp.int32)
            dsp = plsc.load_gather(dinvv, [n16])
            bsp = plsc.load_gather(batv, [jnp.full((16,), ci, jnp.int32),
                                          jnp.full((16,), i, jnp.int32)])
            for q in range(H // 16):
                sl = pl.ds(q * 16, 16)
                v = dsp * (p0v[n, sl] + p1v[n, sl] + hpv[n, sl]) + b4v[sl]
                v = jnp.maximum(v, 0.0)
                rowsb[i, sl] = v
                col = colbase + (q * 16)
                cur = plsc.load_gather(maxp, [bsp, col])
                plsc.store_scatter(maxp, [bsp, col], jnp.maximum(cur, v))
            curc = plsc.load_gather(cntp, [zi16, bsp])
            plsc.store_scatter(cntp, [zi16, bsp], curc + one16)
            return 0

        lax.fori_loop(0, 64, node_body, 0)
        pltpu.sync_copy(rowsb, sacc.at[batv.at[ci]], add=True)
        return 0

    lax.fori_loop(0, NODES_PER_TILE // 64, chunk_body, 0)
    plsc.subcore_barrier()

    pltpu.sync_copy(maxp, max_hbm.at[wid])
    pltpu.sync_copy(cntp, cnt_hbm.at[wid])
    pltpu.sync_copy(sacc.at[pl.ds(s * BROWS, BROWS)],
                    sum_hbm.at[c, pl.ds(s * BROWS, BROWS)])


_ROWBLK = 1024
_NBLK = NPAD // _ROWBLK


def _tc_pre_body(x_ref, degp_ref, w_ref, hp_ref, dinv_ref):
    d = degp_ref[0, :, 0:1] + degp_ref[1, :, 0:1] + 1.0
    dv = lax.rsqrt(d)
    h = jnp.dot(x_ref[...], w_ref[...], precision="highest",
                preferred_element_type=jnp.float32)
    hp_ref[...] = h * dv
    dinv_ref[...] = dv


def _tc_pre(x, degp, w1):
    return pl.pallas_call(
        _tc_pre_body,
        grid=(_NBLK,),
        in_specs=[
            pl.BlockSpec((_ROWBLK, DIN), lambda i: (i, 0)),
            pl.BlockSpec((NC, _ROWBLK, DW), lambda i: (0, i, 0)),
            pl.BlockSpec((DIN, H), lambda i: (0, 0)),
        ],
        out_specs=[
            pl.BlockSpec((_ROWBLK, H), lambda i: (i, 0)),
            pl.BlockSpec((_ROWBLK, 1), lambda i: (i, 0)),
        ],
        out_shape=[
            jax.ShapeDtypeStruct((NPAD, H), jnp.float32),
            jax.ShapeDtypeStruct((NPAD, 1), jnp.float32),
        ],
    )(x, degp, w1)


def _tc_mid_body(p_ref, hp_ref, dinv_ref, b_ref, w_ref, out_ref):
    dv = dinv_ref[...]
    t = dv * (p_ref[0] + p_ref[1] + hp_ref[...]) + b_ref[...]
    t = jnp.maximum(t, 0.0)
    out_ref[...] = jnp.dot(t, w_ref[...], precision="highest",
                           preferred_element_type=jnp.float32) * dv


def _tc_mid(parts, hp, dinv, b, w):
    return pl.pallas_call(
        _tc_mid_body,
        grid=(_NBLK,),
        in_specs=[
            pl.BlockSpec((NC, _ROWBLK, H), lambda i: (0, i, 0)),
            pl.BlockSpec((_ROWBLK, H), lambda i: (i, 0)),
            pl.BlockSpec((_ROWBLK, 1), lambda i: (i, 0)),
            pl.BlockSpec((1, H), lambda i: (0, 0)),
            pl.BlockSpec((H, H), lambda i: (0, 0)),
        ],
        out_specs=pl.BlockSpec((_ROWBLK, H), lambda i: (i, 0)),
        out_shape=jax.ShapeDtypeStruct((NPAD, H), jnp.float32),
    )(parts, hp, dinv, b, w)


def _tc_head_body(sum_ref, max_ref, cnt_ref, wo_ref, bo_ref, out_ref, pooled_ref):
    ssum = sum_ref[0, :B, :] + sum_ref[1, :B, :]
    mx = jnp.max(max_ref[...], axis=0)[:B, :]
    cnt = jnp.sum(cnt_ref[...], axis=0)[:B, :]
    mean = ssum / jnp.maximum(cnt, 1.0)
    pooled = jnp.concatenate([mean, mx], axis=1)
    out_ref[...] = jnp.dot(pooled, wo_ref[...], precision="highest",
                           preferred_element_type=jnp.float32) + bo_ref[...]
    pooled_ref[...] = pooled


def _tc_head(sump, maxp, cntp, w_out, b_out):
    return pl.pallas_call(
        _tc_head_body,
        in_specs=[
            pl.BlockSpec((NC, BPAD, H), lambda: (0, 0, 0)),
            pl.BlockSpec((NW, BPAD, H), lambda: (0, 0, 0)),
            pl.BlockSpec((NW, BPAD, 1), lambda: (0, 0, 0)),
            pl.BlockSpec((2 * H, 1), lambda: (0, 0)),
            pl.BlockSpec((1, 1), lambda: (0, 0)),
        ],
        out_specs=[
            pl.BlockSpec((B, 1), lambda: (0, 0)),
            pl.BlockSpec((B, 2 * H), lambda: (0, 0)),
        ],
        out_shape=[
            jax.ShapeDtypeStruct((B, 1), jnp.float32),
            jax.ShapeDtypeStruct((B, 2 * H), jnp.float32),
        ],
    )(sump, maxp, cntp, w_out, b_out)


def _edges_3d(flat):
    nf = NS * CPT_F * CH
    a = flat[:nf].reshape(NS, CPT_F, CH)
    b = flat[nf:].reshape(NS, CPT_S, CH)
    b = jnp.pad(b, ((0, 0), (0, CPTMAX - CPT_S), (0, 0)), constant_values=N)
    if FAST_CORE == 0:
        return jnp.concatenate([a, b], axis=0)
    return jnp.concatenate([b, a], axis=0)


def kernel(x, edge_index, batch_index, W1, b1, W2, b2, W3, b3, W4, b4, W_out, b_out):
    src = edge_index[0]
    dst = edge_index[1]
    pad_e = jnp.full((EPAD - E,), N, jnp.int32)
    src3d = _edges_3d(jnp.concatenate([src, pad_e]))
    dst3d = _edges_3d(jnp.concatenate([dst, pad_e]))
    batch3d = jnp.concatenate(
        [batch_index, jnp.full((NPAD - N,), B, jnp.int32)]
    ).reshape(NW, NODES_PER_TILE // 64, 64)
    x_pad = jnp.concatenate([x, jnp.zeros((NPAD - N, DIN), x.dtype)], axis=0)

    degp = _sc_deg(dst3d)
    hp, dinv = _tc_pre(x_pad, degp, W1)
    dinv_flat = dinv.reshape(NPAD)

    for (bb, ww) in ((b1, W2), (b2, W3), (b3, W4)):
        parts = _sc_agg(hp, src3d, dst3d)
        hp = _tc_mid(parts, hp, dinv, bb.reshape(1, H), ww)
    parts = _sc_agg(hp, src3d, dst3d)

    sump, maxp, cntp = _sc_pool(parts, hp, dinv_flat, batch3d, b4)
    out, pooled = _tc_head(sump, maxp, cntp.reshape(NW, BPAD, 1),
                           W_out, b_out.reshape(1, 1))
    return (out, pooled)


# speedup vs baseline: 1.4727x; 1.4727x over previous
import functools

import jax
import jax.numpy as jnp
from jax import lax
from jax.experimental import pallas as pl
from jax.experimental.pallas import tpu as pltpu
from jax.experimental.pallas import tpu_sc as plsc

N = 10000
E = 320000
DIN = 128
H = 64
B = 512

NPAD = 10240
NC, NS = 2, 16
NW = NC * NS
CH = 128
NBUF = 2
CPT = 80
EPAD = NW * CPT * CH
ROWS_PER_TILE = NPAD // NS
NODES_PER_TILE = NPAD // NW
DW = 16
BPAD = 640
BROWS = BPAD // NS

_mesh = plsc.VectorSubcoreMesh(core_axis_name="c", subcore_axis_name="s")


def _zero_vmem_2d(ref, nrows, width):
    z = jnp.zeros((16,), jnp.float32)

    def body(j, _):
        for q in range(width // 16):
            ref[j, pl.ds(q * 16, 16)] = z
        return 0

    lax.fori_loop(0, nrows, body, 0)


@functools.partial(
    pl.kernel,
    out_type=jax.ShapeDtypeStruct((NC, NPAD, DW), jnp.float32),
    mesh=_mesh,
    compiler_params=pltpu.CompilerParams(use_tc_tiling_on_sc=False, needs_layout_passes=False),
    scratch_types=[
        pltpu.VMEM((CPT, CH), jnp.int32),
        pltpu.VMEM((CH, DW), jnp.float32),
        pltpu.VMEM((CH, DW), jnp.float32),
        pltpu.VMEM_SHARED((NPAD, DW), jnp.float32),
    ],
)
def _sc_deg(dst_hbm, out_hbm, dstv, onesv, zbuf, acc):
    c = lax.axis_index("c")
    s = lax.axis_index("s")
    wid = c * NS + s

    onehot = jnp.where(lax.iota(jnp.int32, 16) == 0, 1.0, 0.0).astype(jnp.float32)
    z = jnp.zeros((16,), jnp.float32)

    def init_body(j, _):
        onesv[j, pl.ds(0, 16)] = onehot
        zbuf[j, pl.ds(0, 16)] = z
        return 0

    lax.fori_loop(0, CH, init_body, 0)

    base = s * ROWS_PER_TILE
    for r in range(ROWS_PER_TILE // CH):
        pltpu.sync_copy(zbuf, acc.at[pl.ds(base + r * CH, CH)])
    plsc.subcore_barrier()

    pltpu.sync_copy(dst_hbm.at[wid], dstv)

    def edge_body(j, _):
        pltpu.sync_copy(onesv, acc.at[dstv.at[j]], add=True)
        return 0

    lax.fori_loop(0, CPT, edge_body, 0)
    plsc.subcore_barrier()

    pltpu.sync_copy(acc.at[pl.ds(base, ROWS_PER_TILE)],
                    out_hbm.at[c, pl.ds(base, ROWS_PER_TILE)])


_MASK_HI = -65536
@functools.partial(
    pl.kernel,
    out_type=jax.ShapeDtypeStruct((NC, NPAD, H), jnp.float32),
    mesh=_mesh,
    compiler_params=pltpu.CompilerParams(use_tc_tiling_on_sc=False, needs_layout_passes=False),
    scratch_types=[
        pltpu.VMEM((CPT, CH), jnp.int32),
        pltpu.VMEM((CPT, CH), jnp.int32),
        pltpu.VMEM((NBUF, CH, H // 2), jnp.int32),
        pltpu.VMEM((CH, H), jnp.float32),
        pltpu.VMEM((CH, H), jnp.float32),
        pltpu.VMEM_SHARED((NPAD, H), jnp.float32),
    ] + [pltpu.SemaphoreType.DMA] * NBUF,
)
def _sc_agg(hp_hbm, src_hbm, dst_hbm, out_hbm, srcv, dstv, rowsp, rowsf,
            zbuf, acc, *sems):
    c = lax.axis_index("c")
    s = lax.axis_index("s")
    wid = c * NS + s

    _zero_vmem_2d(zbuf, CH, H)
    base = s * ROWS_PER_TILE
    for r in range(ROWS_PER_TILE // CH):
        pltpu.sync_copy(zbuf, acc.at[pl.ds(base + r * CH, CH)])
    plsc.subcore_barrier()

    pltpu.sync_copy(src_hbm.at[wid], srcv)
    pltpu.sync_copy(dst_hbm.at[wid], dstv)

    def convert(b):
        def conv_body(i, _):
            for q in range(H // 32):
                v = rowsp[b, i, pl.ds(q * 16, 16)]
                lo = lax.bitcast_convert_type(v << 16, jnp.float32)
                hi = lax.bitcast_convert_type(v & _MASK_HI, jnp.float32)
                rowsf[i, pl.ds(q * 32, 16)] = lo
                rowsf[i, pl.ds(q * 32 + 16, 16)] = hi
            return 0

        lax.fori_loop(0, CH, conv_body, 0)

    for b in range(NBUF):
        pltpu.async_copy(hp_hbm.at[srcv.at[b]], rowsp.at[b], sems[b])

    def group_body(g, _):
        for b in range(NBUF):
            j = g * NBUF + b
            pltpu.make_async_copy(hp_hbm.at[srcv.at[j]], rowsp.at[b],
                                  sems[b]).wait()
            convert(b)
            pltpu.async_copy(hp_hbm.at[srcv.at[j + NBUF]], rowsp.at[b],
                             sems[b])
            pltpu.sync_copy(rowsf, acc.at[dstv.at[j]], add=True)
        return 0

    lax.fori_loop(0, CPT // NBUF - 1, group_body, 0)
    for b in range(NBUF):
        j = CPT - NBUF + b
        pltpu.make_async_copy(hp_hbm.at[srcv.at[j]], rowsp.at[b],
                              sems[b]).wait()
        convert(b)
        pltpu.sync_copy(rowsf, acc.at[dstv.at[j]], add=True)

    plsc.subcore_barrier()

    pltpu.sync_copy(acc.at[pl.ds(base, ROWS_PER_TILE)],
                    out_hbm.at[c, pl.ds(base, ROWS_PER_TILE)])


@functools.partial(
    pl.kernel,
    out_type=(
        jax.ShapeDtypeStruct((NC, BPAD, H), jnp.float32),
        jax.ShapeDtypeStruct((NW, BPAD, H), jnp.float32),
        jax.ShapeDtypeStruct((NW, 1, BPAD), jnp.float32),
    ),
    mesh=_mesh,
    compiler_params=pltpu.CompilerParams(use_tc_tiling_on_sc=False, needs_layout_passes=False),
    scratch_types=[
        pltpu.VMEM((NODES_PER_TILE, H), jnp.float32),
        pltpu.VMEM((NODES_PER_TILE, H), jnp.float32),
        pltpu.VMEM((NODES_PER_TILE, H), jnp.float32),
        pltpu.VMEM((NODES_PER_TILE,), jnp.float32),
        pltpu.VMEM((NODES_PER_TILE // 64, 64), jnp.int32),
        pltpu.VMEM((H,), jnp.float32),
        pltpu.VMEM((64, H), jnp.float32),
        pltpu.VMEM((BPAD, H), jnp.float32),
        pltpu.VMEM((1, BPAD), jnp.float32),
        pltpu.VMEM_SHARED((BPAD, H), jnp.float32),
    ],
)
def _sc_pool(p_hbm, hp_hbm, dinv_hbm, batch_hbm, b4_hbm,
             sum_hbm, max_hbm, cnt_hbm,
             p0v, p1v, hpv, dinvv, batv, b4v, rowsb, maxp, cntp, sacc):
    c = lax.axis_index("c")
    s = lax.axis_index("s")
    wid = c * NS + s
    nbase = wid * NODES_PER_TILE

    _zero_vmem_2d(maxp, BPAD, H)
    z16 = jnp.zeros((16,), jnp.float32)
    for q in range(BPAD // 16):
        cntp[0, pl.ds(q * 16, 16)] = z16
    _zero_vmem_2d(rowsb, 64, H)
    pltpu.sync_copy(rowsb.at[pl.ds(0, BROWS)], sacc.at[pl.ds(s * BROWS, BROWS)])
    plsc.subcore_barrier()

    pltpu.sync_copy(p_hbm.at[0, pl.ds(nbase, NODES_PER_TILE)], p0v)
    pltpu.sync_copy(p_hbm.at[1, pl.ds(nbase, NODES_PER_TILE)], p1v)
    pltpu.sync_copy(hp_hbm.at[pl.ds(nbase, NODES_PER_TILE)], hpv)
    pltpu.sync_copy(dinv_hbm.at[pl.ds(nbase, NODES_PER_TILE)], dinvv)
    pltpu.sync_copy(batch_hbm.at[wid], batv)
    pltpu.sync_copy(b4_hbm, b4v)

    colbase = lax.iota(jnp.int32, 16)
    one16 = jnp.full((16,), 1.0, jnp.float32)
    zi16 = jnp.zeros((16,), jnp.int32)

    def chunk_body(ci, _):
        def node_body(i, _):
            n = ci * 64 + i
            n16 = jnp.full((16,), n, jnp.int32)
            dsp = plsc.load_gather(dinvv, [n16])
            bsp = plsc.load_gather(batv, [jnp.full((16,), ci, jnp.int32),
                                          jnp.full((16,), i, jnp.int32)])
            for q in range(H // 16):
                sl = pl.ds(q * 16, 16)
                v = dsp * (p0v[n, sl] + p1v[n, sl] + hpv[n, sl]) + b4v[sl]
                v = jnp.maximum(v, 0.0)
                rowsb[i, sl] = v
                col = colbase + (q * 16)
                cur = plsc.load_gather(maxp, [bsp, col])
                plsc.store_scatter(maxp, [bsp, col], jnp.maximum(cur, v))
            curc = plsc.load_gather(cntp, [zi16, bsp])
            plsc.store_scatter(cntp, [zi16, bsp], curc + one16)
            return 0

        lax.fori_loop(0, 64, node_body, 0)
        pltpu.sync_copy(rowsb, sacc.at[batv.at[ci]], add=True)
        return 0

    lax.fori_loop(0, NODES_PER_TILE // 64, chunk_body, 0)
    plsc.subcore_barrier()

    pltpu.sync_copy(maxp, max_hbm.at[wid])
    pltpu.sync_copy(cntp, cnt_hbm.at[wid])
    pltpu.sync_copy(sacc.at[pl.ds(s * BROWS, BROWS)],
                    sum_hbm.at[c, pl.ds(s * BROWS, BROWS)])


_ROWBLK = 1024
_NBLK = NPAD // _ROWBLK


def _pack_rows(hp):
    bits = lax.bitcast_convert_type(hp, jnp.int32)
    r = bits + (0x7FFF + (lax.shift_right_logical(bits, 16) & 1))
    blocks = []
    for q in range(H // 32):
        rlo = r[:, 32 * q:32 * q + 16]
        rhi = r[:, 32 * q + 16:32 * q + 32]
        blocks.append((rhi & _MASK_HI) |
                      (lax.shift_right_logical(rlo, 16) & 0xFFFF))
    return jnp.concatenate(blocks, axis=1)


def _tc_pre_body(x_ref, degp_ref, w_ref, hp_ref, hpp_ref, dinv_ref):
    d = degp_ref[0, :, 0:1] + degp_ref[1, :, 0:1] + 1.0
    dv = lax.rsqrt(d)
    h = jnp.dot(x_ref[...], w_ref[...], precision="highest",
                preferred_element_type=jnp.float32)
    hp = h * dv
    hp_ref[...] = hp
    hpp_ref[...] = _pack_rows(hp)
    dinv_ref[...] = dv


def _tc_pre(x, degp, w1):
    return pl.pallas_call(
        _tc_pre_body,
        grid=(_NBLK,),
        in_specs=[
            pl.BlockSpec((_ROWBLK, DIN), lambda i: (i, 0)),
            pl.BlockSpec((NC, _ROWBLK, DW), lambda i: (0, i, 0)),
            pl.BlockSpec((DIN, H), lambda i: (0, 0)),
        ],
        out_specs=[
            pl.BlockSpec((_ROWBLK, H), lambda i: (i, 0)),
            pl.BlockSpec((_ROWBLK, H // 2), lambda i: (i, 0)),
            pl.BlockSpec((_ROWBLK, 1), lambda i: (i, 0)),
        ],
        out_shape=[
            jax.ShapeDtypeStruct((NPAD, H), jnp.float32),
            jax.ShapeDtypeStruct((NPAD, H // 2), jnp.int32),
            jax.ShapeDtypeStruct((NPAD, 1), jnp.float32),
        ],
    )(x, degp, w1)


def _tc_mid_body(p_ref, hp_ref, dinv_ref, b_ref, w_ref, out_ref, outp_ref):
    dv = dinv_ref[...]
    t = dv * (p_ref[0] + p_ref[1] + hp_ref[...]) + b_ref[...]
    t = jnp.maximum(t, 0.0)
    o = jnp.dot(t, w_ref[...], precision="highest",
                preferred_element_type=jnp.float32) * dv
    out_ref[...] = o
    outp_ref[...] = _pack_rows(o)


def _tc_mid(parts, hp, dinv, b, w):
    return pl.pallas_call(
        _tc_mid_body,
        grid=(_NBLK,),
        in_specs=[
            pl.BlockSpec((NC, _ROWBLK, H), lambda i: (0, i, 0)),
            pl.BlockSpec((_ROWBLK, H), lambda i: (i, 0)),
            pl.BlockSpec((_ROWBLK, 1), lambda i: (i, 0)),
            pl.BlockSpec((1, H), lambda i: (0, 0)),
            pl.BlockSpec((H, H), lambda i: (0, 0)),
        ],
        out_specs=[
            pl.BlockSpec((_ROWBLK, H), lambda i: (i, 0)),
            pl.BlockSpec((_ROWBLK, H // 2), lambda i: (i, 0)),
        ],
        out_shape=[
            jax.ShapeDtypeStruct((NPAD, H), jnp.float32),
            jax.ShapeDtypeStruct((NPAD, H // 2), jnp.int32),
        ],
    )(parts, hp, dinv, b, w)


def _tc_head_body(sum_ref, max_ref, cnt_ref, wo_ref, bo_ref, out_ref, pooled_ref):
    ssum = sum_ref[0, :B, :] + sum_ref[1, :B, :]
    mx = jnp.max(max_ref[...], axis=0)[:B, :]
    cnt = jnp.sum(cnt_ref[...], axis=0)[:B, :]
    mean = ssum / jnp.maximum(cnt, 1.0)
    pooled = jnp.concatenate([mean, mx], axis=1)
    out_ref[...] = jnp.dot(pooled, wo_ref[...], precision="highest",
                           preferred_element_type=jnp.float32) + bo_ref[...]
    pooled_ref[...] = pooled


def _tc_head(sump, maxp, cntp, w_out, b_out):
    return pl.pallas_call(
        _tc_head_body,
        in_specs=[
            pl.BlockSpec((NC, BPAD, H), lambda: (0, 0, 0)),
            pl.BlockSpec((NW, BPAD, H), lambda: (0, 0, 0)),
            pl.BlockSpec((NW, BPAD, 1), lambda: (0, 0, 0)),
            pl.BlockSpec((2 * H, 1), lambda: (0, 0)),
            pl.BlockSpec((1, 1), lambda: (0, 0)),
        ],
        out_specs=[
            pl.BlockSpec((B, 1), lambda: (0, 0)),
            pl.BlockSpec((B, 2 * H), lambda: (0, 0)),
        ],
        out_shape=[
            jax.ShapeDtypeStruct((B, 1), jnp.float32),
            jax.ShapeDtypeStruct((B, 2 * H), jnp.float32),
        ],
    )(sump, maxp, cntp, w_out, b_out)


def kernel(x, edge_index, batch_index, W1, b1, W2, b2, W3, b3, W4, b4, W_out, b_out):
    src = edge_index[0]
    dst = edge_index[1]
    pad_e = jnp.full((EPAD - E,), N, jnp.int32)
    src3d = jnp.concatenate([src, pad_e]).reshape(NW, CPT, CH)
    dst3d = jnp.concatenate([dst, pad_e]).reshape(NW, CPT, CH)
    batch3d = jnp.concatenate(
        [batch_index, jnp.full((NPAD - N,), B, jnp.int32)]
    ).reshape(NW, NODES_PER_TILE // 64, 64)
    x_pad = jnp.concatenate([x, jnp.zeros((NPAD - N, DIN), x.dtype)], axis=0)

    degp = _sc_deg(dst3d)
    hp, hpp, dinv = _tc_pre(x_pad, degp, W1)
    dinv_flat = dinv.reshape(NPAD)

    for (bb, ww) in ((b1, W2), (b2, W3), (b3, W4)):
        parts = _sc_agg(hpp, src3d, dst3d)
        hp, hpp = _tc_mid(parts, hp, dinv, bb.reshape(1, H), ww)
    parts = _sc_agg(hpp, src3d, dst3d)

    sump, maxp, cntp = _sc_pool(parts, hp, dinv_flat, batch3d, b4)
    out, pooled = _tc_head(sump, maxp, cntp.reshape(NW, BPAD, 1),
                           W_out, b_out.reshape(1, 1))
    return (out, pooled)
